# baseline (speedup 1.0000x reference)
import functools

import jax
import jax.numpy as jnp
from jax import lax
from jax.experimental import pallas as pl
from jax.experimental.pallas import tpu as pltpu
from jax.experimental.pallas import tpu_sc as plsc

B = 4096
EMB = 64
HID = 64
NCLS = 1000
LANES = 128


def _make_sc_gather(V: int, Bn: int):
    info = plsc.get_sparse_core_info()
    NC, NS = info.num_cores, info.num_subcores
    NW = NC * NS
    assert Bn % NW == 0
    b_per_w = Bn // NW
    RT = 4
    n_rounds = b_per_w // RT
    mesh = plsc.VectorSubcoreMesh(core_axis_name="c", subcore_axis_name="s")

    @functools.partial(
        pl.kernel,
        mesh=mesh,
        out_type=jax.ShapeDtypeStruct((EMB, Bn), jnp.float32),
        scratch_types=[
            pltpu.VMEM((b_per_w,), jnp.int32),
            pltpu.VMEM((EMB, RT * LANES), jnp.float32),
            pltpu.VMEM((EMB, RT * LANES), jnp.float32),
            pltpu.VMEM((EMB, RT * LANES), jnp.float32),
            pltpu.VMEM((EMB, b_per_w), jnp.float32),
            pltpu.SemaphoreType.DMA,
        ],
        compiler_params=pltpu.CompilerParams(needs_layout_passes=False),
    )
    def gather_kernel(
        tt_hbm, idx_hbm, out_hbm, idx_v, stg0_v, stg1_v, stg2_v, po_v, sem
    ):
        wid = lax.axis_index("s") * NC + lax.axis_index("c")
        base = wid * b_per_w
        pltpu.sync_copy(idx_hbm.at[pl.ds(base, b_per_w)], idx_v)
        iota = lax.iota(jnp.int32, 16)
        stgs = [stg0_v, stg1_v, stg2_v]

        def fetch(r):
            idv = idx_v[pl.ds((r * RT // 16) * 16, 16)]
            stg = stgs[r % 3]
            descs = []
            for t in range(RT):
                j = r * RT + t
                cid = idv[j % 16] >> 7
                descs.append(
                    pltpu.async_copy(
                        tt_hbm.at[
                            :, pl.ds(pl.multiple_of(cid * LANES, LANES), LANES)
                        ],
                        stg.at[:, pl.ds(t * LANES, LANES)],
                        sem,
                    )
                )
            return descs

        def extract(r, descs):
            for d in descs:
                d.wait()
            idv = idx_v[pl.ds((r * RT // 16) * 16, 16)]
            stg = stgs[r % 3]
            for t in range(RT):
                j = r * RT + t
                lane = (idv[j % 16] & (LANES - 1)) + t * LANES
                lane_vec = jnp.full((16,), lane, jnp.int32)
                col_vec = jnp.full((16,), j, jnp.int32)
                for c in range(EMB // 16):
                    feat = c * 16 + iota
                    x = plsc.load_gather(stg, [feat, lane_vec])
                    plsc.store_scatter(po_v, [feat, col_vec], x)

        pend = [fetch(0), fetch(1)]
        for r in range(2, n_rounds):
            nxt = fetch(r)
            extract(r - 2, pend[0])
            pend = [pend[1], nxt]
        extract(n_rounds - 2, pend[0])
        extract(n_rounds - 1, pend[1])
        pltpu.sync_copy(po_v, out_hbm.at[:, pl.ds(base, b_per_w)])

    return gather_kernel


def _mlp_body(pt_ref, w1_ref, b1t_ref, w2_ref, b2t_ref, out_ref):
    h = jnp.maximum(
        lax.dot_general(
            w1_ref[...], pt_ref[...], (((0,), (0,)), ((), ())),
            preferred_element_type=jnp.float32,
        )
        + b1t_ref[...],
        0.0,
    )
    out_ref[...] = (
        lax.dot_general(
            w2_ref[...], h, (((0,), (0,)), ((), ())),
            preferred_element_type=jnp.float32,
        )
        + b2t_ref[...]
    )


def _mlp(pooledT, W1, b1, W2, b2):
    BLK = 2048
    grid = (B // BLK,)
    return pl.pallas_call(
        _mlp_body,
        grid=grid,
        in_specs=[
            pl.BlockSpec((EMB, BLK), lambda i: (0, i)),
            pl.BlockSpec((EMB, HID), lambda i: (0, 0)),
            pl.BlockSpec((HID, 1), lambda i: (0, 0)),
            pl.BlockSpec((HID, NCLS), lambda i: (0, 0)),
            pl.BlockSpec((NCLS, 1), lambda i: (0, 0)),
        ],
        out_specs=pl.BlockSpec((NCLS, BLK), lambda i: (0, i)),
        out_shape=jax.ShapeDtypeStruct((NCLS, B), jnp.float32),
    )(pooledT, W1, b1.reshape(HID, 1), W2, b2.reshape(NCLS, 1))


def kernel(input_ids, offsets, table, W1, b1, W2, b2):
    del offsets
    ids = input_ids.astype(jnp.int32)
    tt = table.T
    gather = _make_sc_gather(tt.shape[1], B)
    pooledT = gather(tt, ids)
    return _mlp(pooledT, W1, b1, W2, b2).T


# speedup vs baseline: 1.0095x; 1.0095x over previous
import functools

import jax
import jax.numpy as jnp
from jax import lax
from jax.experimental import pallas as pl
from jax.experimental.pallas import tpu as pltpu
from jax.experimental.pallas import tpu_sc as plsc

B = 4096
EMB = 64
HID = 64
NCLS = 1000
LANES = 128


def _make_sc_gather(V: int, Bn: int):
    info = plsc.get_sparse_core_info()
    NC, NS = info.num_cores, info.num_subcores
    NW = NC * NS
    assert Bn % NW == 0
    b_per_w = Bn // NW
    RT = 4
    n_rounds = b_per_w // RT
    mesh = plsc.VectorSubcoreMesh(core_axis_name="c", subcore_axis_name="s")

    @functools.partial(
        pl.kernel,
        mesh=mesh,
        out_type=jax.ShapeDtypeStruct((EMB, Bn), jnp.float32),
        scratch_types=[
            pltpu.VMEM((b_per_w,), jnp.int32),
            pltpu.VMEM((EMB, RT * LANES), jnp.float32),
            pltpu.VMEM((EMB, RT * LANES), jnp.float32),
            pltpu.VMEM((EMB, b_per_w), jnp.float32),
            pltpu.SemaphoreType.DMA,
        ],
        compiler_params=pltpu.CompilerParams(needs_layout_passes=False),
    )
    def gather_kernel(tt_hbm, idx_hbm, out_hbm, idx_v, stg0_v, stg1_v, po_v, sem):
        wid = lax.axis_index("s") * NC + lax.axis_index("c")
        base = wid * b_per_w
        pltpu.sync_copy(idx_hbm.at[pl.ds(base, b_per_w)], idx_v)
        iota = lax.iota(jnp.int32, 16)
        stgs = [stg0_v, stg1_v]

        def fetch(r):
            idv = idx_v[pl.ds((r * RT // 16) * 16, 16)]
            stg = stgs[r % 2]
            descs = []
            for t in range(RT):
                j = r * RT + t
                cid = idv[j % 16] >> 7
                descs.append(
                    pltpu.async_copy(
                        tt_hbm.at[
                            :, pl.ds(pl.multiple_of(cid * LANES, LANES), LANES)
                        ],
                        stg.at[:, pl.ds(t * LANES, LANES)],
                        sem,
                    )
                )
            return descs

        def extract(r, descs):
            for d in descs:
                d.wait()
            idv = idx_v[pl.ds((r * RT // 16) * 16, 16)]
            stg = stgs[r % 2]
            for t in range(RT):
                j = r * RT + t
                lane = (idv[j % 16] & (LANES - 1)) + t * LANES
                lane_vec = jnp.full((16,), lane, jnp.int32)
                col_vec = jnp.full((16,), j, jnp.int32)
                for c in range(EMB // 16):
                    feat = c * 16 + iota
                    x = plsc.load_gather(stg, [feat, lane_vec])
                    plsc.store_scatter(po_v, [feat, col_vec], x)

        pend = fetch(0)
        for r in range(1, n_rounds):
            nxt = fetch(r)
            extract(r - 1, pend)
            pend = nxt
        extract(n_rounds - 1, pend)
        pltpu.sync_copy(po_v, out_hbm.at[:, pl.ds(base, b_per_w)])

    return gather_kernel


def _mlp_body(pt_ref, w1_ref, b1t_ref, w2_ref, b2t_ref, out_ref):
    h = jnp.maximum(
        lax.dot_general(
            w1_ref[...], pt_ref[...], (((0,), (0,)), ((), ())),
            preferred_element_type=jnp.float32,
        )
        + b1t_ref[...],
        0.0,
    )
    out_ref[...] = (
        lax.dot_general(
            w2_ref[...], h, (((0,), (0,)), ((), ())),
            preferred_element_type=jnp.float32,
        )
        + b2t_ref[...]
    )


def _mlp(pooledT, W1, b1, W2, b2):
    BLK = 2048
    grid = (B // BLK,)
    return pl.pallas_call(
        _mlp_body,
        grid=grid,
        in_specs=[
            pl.BlockSpec((EMB, BLK), lambda i: (0, i)),
            pl.BlockSpec((EMB, HID), lambda i: (0, 0)),
            pl.BlockSpec((HID, 1), lambda i: (0, 0)),
            pl.BlockSpec((HID, NCLS), lambda i: (0, 0)),
            pl.BlockSpec((NCLS, 1), lambda i: (0, 0)),
        ],
        out_specs=pl.BlockSpec((NCLS, BLK), lambda i: (0, i)),
        out_shape=jax.ShapeDtypeStruct((NCLS, B), jnp.float32),
    )(pooledT, W1, b1.reshape(HID, 1), W2, b2.reshape(NCLS, 1))


def kernel(input_ids, offsets, table, W1, b1, W2, b2):
    del offsets
    ids = input_ids.astype(jnp.int32)
    tt = table.T
    gather = _make_sc_gather(tt.shape[1], B)
    pooledT = gather(tt, ids)
    return _mlp(pooledT, W1, b1, W2, b2).T
